# baseline (speedup 1.0000x reference)
import jax
import jax.numpy as jnp
from jax.experimental import pallas as pl

B, N, D_NODE = 16, 128, 128
HID1, HID2 = 128, 256
EDGE_HID = 128


def _gnn_kernel(nf_ref, ef_ref, W1_ref, b1_ref, W2_ref, b2_ref,
                We_ref, be_ref, Wc_ref, bc_ref, out_ref):
    ef = ef_ref[...].reshape(B, -1)
    m = jnp.mean(nf_ref[...], axis=1)
    e1 = jax.nn.relu(
        jnp.dot(m, W1_ref[...], preferred_element_type=jnp.float32)
        + b1_ref[...])
    e2 = jax.nn.relu(
        jnp.dot(e1, W2_ref[...], preferred_element_type=jnp.float32)
        + b2_ref[...])
    ee = jax.nn.relu(
        jnp.dot(ef, We_ref[...], preferred_element_type=jnp.float32)
        + be_ref[...])
    out_ref[...] = (
        jnp.dot(e2, Wc_ref[0:HID2, :], preferred_element_type=jnp.float32)
        + jnp.dot(ee, Wc_ref[HID2:HID2 + EDGE_HID, :],
                  preferred_element_type=jnp.float32)
        + bc_ref[...])


def kernel(node_features, edge_features, W1, b1, W2, b2, We, be, Wc, bc):
    b = node_features.shape[0]
    out = pl.pallas_call(
        _gnn_kernel,
        out_shape=jax.ShapeDtypeStruct((b, Wc.shape[1]), jnp.float32),
    )(node_features, edge_features,
      W1, b1.reshape(1, -1), W2, b2.reshape(1, -1),
      We, be.reshape(1, -1),
      Wc, bc.reshape(1, -1))
    return out


# speedup vs baseline: 1.2169x; 1.2169x over previous
import jax
import jax.numpy as jnp
from jax.experimental import pallas as pl

B, N, D_NODE = 16, 128, 128
HID1, HID2 = 128, 256
EDGE_HID = 128


def _gnn_kernel(nf_ref, ef_ref, W1_ref, b1_ref, W2_ref, b2_ref,
                We_ref, be_ref, Wc_ref, bc_ref, out_ref):
    m = jnp.mean(nf_ref[...], axis=1)
    e1 = jax.nn.relu(
        jnp.dot(m, W1_ref[...], preferred_element_type=jnp.float32)
        + b1_ref[...])
    e2 = jax.nn.relu(
        jnp.dot(e1, W2_ref[...], preferred_element_type=jnp.float32)
        + b2_ref[...])
    ee = jax.nn.relu(
        jnp.dot(ef_ref[...], We_ref[...], preferred_element_type=jnp.float32)
        + be_ref[...])
    out_ref[...] = (
        jnp.dot(e2, Wc_ref[0:HID2, :], preferred_element_type=jnp.float32)
        + jnp.dot(ee, Wc_ref[HID2:HID2 + EDGE_HID, :],
                  preferred_element_type=jnp.float32)
        + bc_ref[...])


def kernel(node_features, edge_features, W1, b1, W2, b2, We, be, Wc, bc):
    b = node_features.shape[0]
    ef_flat = edge_features.reshape(b, -1)
    out = pl.pallas_call(
        _gnn_kernel,
        out_shape=jax.ShapeDtypeStruct((b, Wc.shape[1]), jnp.float32),
    )(node_features, ef_flat,
      W1, b1, W2, b2,
      We, be,
      Wc, bc)
    return out


# speedup vs baseline: 1.2958x; 1.0649x over previous
import jax
import jax.numpy as jnp
from jax.experimental import pallas as pl
from jax.experimental.pallas import tpu as pltpu

B, N, D_NODE = 16, 128, 128
HID1, HID2 = 128, 256
EDGE_HID = 128
NK = 2
KC = 12800 // NK


def _gnn_kernel(nf_ref, ef_ref, W1_ref, b1_ref, W2_ref, b2_ref,
                We_ref, be_ref, Wc_ref, bc_ref, out_ref, acc_ref, e2_ref):
    k = pl.program_id(0)
    partial = jnp.dot(ef_ref[...], We_ref[...],
                      preferred_element_type=jnp.float32)

    @pl.when(k == 0)
    def _first():
        acc_ref[...] = partial
        m = jnp.mean(nf_ref[...], axis=1)
        e1 = jax.nn.relu(
            jnp.dot(m, W1_ref[...], preferred_element_type=jnp.float32)
            + b1_ref[...])
        e2_ref[...] = jax.nn.relu(
            jnp.dot(e1, W2_ref[...], preferred_element_type=jnp.float32)
            + b2_ref[...])

    @pl.when(k == NK - 1)
    def _last():
        ee = jax.nn.relu(acc_ref[...] + partial + be_ref[...])
        out_ref[...] = (
            jnp.dot(e2_ref[...], Wc_ref[0:HID2, :],
                    preferred_element_type=jnp.float32)
            + jnp.dot(ee, Wc_ref[HID2:HID2 + EDGE_HID, :],
                      preferred_element_type=jnp.float32)
            + bc_ref[...])


def kernel(node_features, edge_features, W1, b1, W2, b2, We, be, Wc, bc):
    b = node_features.shape[0]
    ef_flat = edge_features.reshape(b, -1)
    full = lambda *shape: pl.BlockSpec(shape, lambda k: (0,) * len(shape))
    out = pl.pallas_call(
        _gnn_kernel,
        grid=(NK,),
        in_specs=[
            full(b, N, D_NODE),
            pl.BlockSpec((b, KC), lambda k: (0, k)),
            full(D_NODE, HID1), full(HID1,),
            full(HID1, HID2), full(HID2,),
            pl.BlockSpec((KC, EDGE_HID), lambda k: (k, 0)),
            full(EDGE_HID,),
            full(HID2 + EDGE_HID, Wc.shape[1]),
            full(Wc.shape[1],),
        ],
        out_specs=pl.BlockSpec((b, Wc.shape[1]), lambda k: (0, 0)),
        out_shape=jax.ShapeDtypeStruct((b, Wc.shape[1]), jnp.float32),
        scratch_shapes=[pltpu.VMEM((b, EDGE_HID), jnp.float32),
                        pltpu.VMEM((b, HID2), jnp.float32)],
        compiler_params=pltpu.CompilerParams(
            dimension_semantics=("arbitrary",)),
    )(node_features, ef_flat,
      W1, b1, W2, b2, We, be, Wc, bc)
    return out
